# Initial kernel scaffold; baseline (speedup 1.0000x reference)
import jax
import jax.numpy as jnp
from jax.experimental import pallas as pl


def kernel(query, query_pos, context, context_pos, W_rff, in_proj_weight, in_proj_bias, out_proj_weight, out_proj_bias):
    raise NotImplementedError("write your pallas kernel here")


# speedup vs baseline: 6.1246x; 6.1246x over previous
import functools
import math

import numpy as np

import jax
import jax.numpy as jnp
from jax import lax
from jax.experimental import pallas as pl
from jax.experimental.pallas import tpu as pltpu
from jax.experimental.pallas import tpu_sc as plsc

B, Q, Z, D = 2, 4096, 4096, 256
H, K = 8, 16
DH = D // H

_TWO_PI = float(np.float32(2.0 * math.pi))
_INV_SQRT_DH = float(np.float32(1.0 / math.sqrt(DH)))

_BZA = 512


def _proj_body(q_ref, qpos_ref, c_ref, cpos_ref, wrff_ref,
               wq_ref, wk_ref, wv_ref, bq_ref, bk_ref, bv_ref,
               qp_ref, kc_ref, vc_ref):
    wr = wrff_ref[...]

    def encode(tok_ref, pos_ref):
        pos = pos_ref[0]
        px = pos[:, 0:1]
        py = pos[:, 1:2]
        proj = _TWO_PI * (px * wr[0:1, :] + py * wr[1:2, :])
        pe = jnp.concatenate([jnp.sin(proj), jnp.cos(proj)], axis=-1)
        return tok_ref[0] + pe

    qe = encode(q_ref, qpos_ref)
    ce = encode(c_ref, cpos_ref)
    dims = (((1,), (1,)), ((), ()))
    qp_ref[0] = lax.dot_general(qe, wq_ref[...], dims,
                                preferred_element_type=jnp.float32) + bq_ref[...]
    kc_ref[0] = lax.dot_general(ce, wk_ref[...], dims,
                                preferred_element_type=jnp.float32) + bk_ref[...]
    vc_ref[0] = lax.dot_general(ce, wv_ref[...], dims,
                                preferred_element_type=jnp.float32) + bv_ref[...]


def _project(query, query_pos, context, context_pos, W_rff, wq, wk, wv, bq, bk, bv):
    grid = (B, Q // _BZA)
    row_spec = pl.BlockSpec((1, _BZA, D), lambda b, i: (b, i, 0))
    pos_spec = pl.BlockSpec((1, _BZA, 2), lambda b, i: (b, i, 0))
    full = lambda shape: pl.BlockSpec(shape, lambda b, i: (0,) * len(shape))
    out_shape = [jax.ShapeDtypeStruct((B, Q, D), jnp.float32) for _ in range(3)]
    return pl.pallas_call(
        _proj_body,
        grid=grid,
        in_specs=[row_spec, pos_spec, row_spec, pos_spec, full((2, D // 2)),
                  full((D, D)), full((D, D)), full((D, D)),
                  full((1, D)), full((1, D)), full((1, D))],
        out_specs=[row_spec, row_spec, row_spec],
        out_shape=out_shape,
    )(query, query_pos, context, context_pos, W_rff, wq, wk, wv, bq, bk, bv)


_BQK = 256


def _knn_body(qpos_t_ref, cpos_ref, qsq_ref, csq_ref, out_ref, dist_ref):
    b = pl.program_id(0)
    bfr = lambda x: x.astype(jnp.bfloat16).astype(jnp.float32)
    qx = bfr(qpos_t_ref[0][0:1, :])
    qy = bfr(qpos_t_ref[0][1:2, :])
    cx = bfr(cpos_ref[0][:, 0:1])
    cy = bfr(cpos_ref[0][:, 1:2])
    cross = qx * cx + qy * cy
    d2 = (qsq_ref[0] + csq_ref[0]) - 2.0 * cross
    dist_ref[...] = jnp.sqrt(jnp.clip(d2, 0.0, None))

    iota0 = lax.broadcasted_iota(jnp.int32, (Z, _BQK), 0)
    rows = []
    for _ in range(K):
        dist = dist_ref[...]
        m = jnp.min(dist, axis=0, keepdims=True)
        idxv = jnp.min(jnp.where(dist == m, iota0, Z),
                       axis=0, keepdims=True)
        rows.append(idxv + b * Z)
        dist_ref[...] = jnp.where(iota0 == idxv, jnp.inf, dist)
    out_ref[0] = jnp.concatenate(rows, axis=0)


def _knn(qpos_t, cpos, qsq, csq):
    grid = (B, Q // _BQK)
    return pl.pallas_call(
        _knn_body,
        grid=grid,
        in_specs=[pl.BlockSpec((1, 2, _BQK), lambda b, i: (b, 0, i)),
                  pl.BlockSpec((1, Z, 2), lambda b, i: (b, 0, 0)),
                  pl.BlockSpec((1, 1, _BQK), lambda b, i: (b, 0, i)),
                  pl.BlockSpec((1, Z, 1), lambda b, i: (b, 0, 0))],
        out_specs=pl.BlockSpec((1, K, _BQK), lambda b, i: (b, 0, i)),
        out_shape=jax.ShapeDtypeStruct((B, K, Q), jnp.int32),
        scratch_shapes=[pltpu.VMEM((Z, _BQK), jnp.float32)],
    )(qpos_t, cpos, qsq, csq)


_NROWS = B * K * Q
_NW = 32
_RPW = _NROWS // _NW
_CHUNK = 128


def _sc_gather(kc2, vc2, idx_flat):
    mesh = plsc.VectorSubcoreMesh(core_axis_name="c", subcore_axis_name="s")

    @functools.partial(
        pl.kernel,
        out_type=[jax.ShapeDtypeStruct((_NROWS, D), jnp.float32),
                  jax.ShapeDtypeStruct((_NROWS, D), jnp.float32)],
        mesh=mesh,
        scratch_types=[pltpu.VMEM((_CHUNK,), jnp.int32),
                       pltpu.VMEM((_CHUNK, D), jnp.float32),
                       pltpu.VMEM((_CHUNK, D), jnp.float32),
                       pltpu.SemaphoreType.DMA,
                       pltpu.SemaphoreType.DMA],
    )
    def gather_kernel(kc_hbm, vc_hbm, idx_hbm, gk_hbm, gv_hbm,
                      idx_v, kbuf, vbuf, sem_k, sem_v):
        wid = lax.axis_index("s") * 2 + lax.axis_index("c")
        base = wid * _RPW

        def body(i, carry):
            off = base + i * _CHUNK
            pltpu.sync_copy(idx_hbm.at[pl.ds(off, _CHUNK)], idx_v)
            ck = pltpu.async_copy(kc_hbm.at[idx_v], kbuf, sem_k)
            cv = pltpu.async_copy(vc_hbm.at[idx_v], vbuf, sem_v)
            ck.wait()
            cv.wait()
            pltpu.sync_copy(kbuf, gk_hbm.at[pl.ds(off, _CHUNK)])
            pltpu.sync_copy(vbuf, gv_hbm.at[pl.ds(off, _CHUNK)])
            return carry

        lax.fori_loop(0, _RPW // _CHUNK, body, 0)

    return gather_kernel(kc2, vc2, idx_flat)


_BQA = 256


def _attn_body(qp_ref, gk_ref, gv_ref, wo_ref, bo_ref, out_ref):
    qpb = qp_ref[0]

    logit_cols = [[] for _ in range(H)]
    for k in range(K):
        prod = qpb * gk_ref[0, k]
        for h in range(H):
            s = jnp.sum(prod[:, h * DH:(h + 1) * DH], axis=1, keepdims=True)
            logit_cols[h].append(s)

    attn = []
    for h in range(H):
        lh = jnp.concatenate(logit_cols[h], axis=1) * _INV_SQRT_DH
        mx = jnp.max(lh, axis=1, keepdims=True)
        e = jnp.exp(lh - mx)
        attn.append(e / jnp.sum(e, axis=1, keepdims=True))

    acc = jnp.zeros((_BQA, D), jnp.float32)
    for k in range(K):
        wk_full = jnp.concatenate(
            [jnp.broadcast_to(attn[h][:, k:k + 1], (_BQA, DH)) for h in range(H)],
            axis=1)
        acc = acc + wk_full * gv_ref[0, k]

    out = lax.dot_general(acc, wo_ref[...], (((1,), (1,)), ((), ())),
                          preferred_element_type=jnp.float32) + bo_ref[...]
    out_ref[0] = out


def _attention(qp, gk, gv, wo, bo):
    grid = (B, Q // _BQA)
    g_spec = pl.BlockSpec((1, K, _BQA, D), lambda b, i: (b, 0, i, 0))
    row_spec = pl.BlockSpec((1, _BQA, D), lambda b, i: (b, i, 0))
    full = lambda shape: pl.BlockSpec(shape, lambda b, i: (0,) * len(shape))
    return pl.pallas_call(
        _attn_body,
        grid=grid,
        in_specs=[row_spec, g_spec, g_spec, full((D, D)), full((1, D))],
        out_specs=row_spec,
        out_shape=jax.ShapeDtypeStruct((B, Q, D), jnp.float32),
    )(qp, gk, gv, wo, bo)


def kernel(query, query_pos, context, context_pos, W_rff,
           in_proj_weight, in_proj_bias, out_proj_weight, out_proj_bias):
    wq = in_proj_weight[:D]
    wk = in_proj_weight[D:2 * D]
    wv = in_proj_weight[2 * D:]
    bq = in_proj_bias[:D].reshape(1, D)
    bk = in_proj_bias[D:2 * D].reshape(1, D)
    bv = in_proj_bias[2 * D:].reshape(1, D)
    bo = out_proj_bias.reshape(1, D)

    qp, kc, vc = _project(query, query_pos, context, context_pos, W_rff,
                          wq, wk, wv, bq, bk, bv)

    qpos_t = query_pos.transpose(0, 2, 1)
    qsq = jnp.sum(query_pos ** 2, axis=-1).reshape(B, 1, Q)
    csq = jnp.sum(context_pos ** 2, axis=-1).reshape(B, Z, 1)
    knn = _knn(qpos_t, context_pos, qsq, csq)

    gk_flat, gv_flat = _sc_gather(kc.reshape(B * Z, D), vc.reshape(B * Z, D),
                                  knn.reshape(_NROWS))
    gk = gk_flat.reshape(B, K, Q, D)
    gv = gv_flat.reshape(B, K, Q, D)

    attn_out = _attention(qp, gk, gv, out_proj_weight, bo)
    return (attn_out, query_pos)
